# Initial kernel scaffold; baseline (speedup 1.0000x reference)
import jax
import jax.numpy as jnp
from jax.experimental import pallas as pl


def kernel(x, edge_index, edge_attr, node2graph, c1_edge_W, c1_edge_b, c1_W1, c1_b1, c1_W2, c1_b2, c1_eps, c2_edge_W, c2_edge_b, c2_W1, c2_b1, c2_W2, c2_b2, c2_eps, gn1_w, gn1_b, gn2_w, gn2_b):
    raise NotImplementedError("write your pallas kernel here")


# speedup vs baseline: 1.9888x; 1.9888x over previous
import functools

import jax
import jax.numpy as jnp
from jax import lax
from jax.experimental import pallas as pl
from jax.experimental.pallas import tpu as pltpu
from jax.experimental.pallas import tpu_sc as plsc

N = 10000
E = 160000
D = 256
DE = 16
G = 64

_P = jax.lax.Precision.HIGHEST
_F32 = jnp.float32

_BE = 2000


def _edge_mlp_body(ea_ref, w_ref, b_ref, out_ref):
    e = jnp.dot(ea_ref[...], w_ref[...], preferred_element_type=_F32,
                precision=_P) + b_ref[...]
    e = e * jax.nn.sigmoid(e)
    out_ref[0] = e[:, :128]
    out_ref[1] = e[:, 128:]


def _edge_mlp(ea, w, b):
    return pl.pallas_call(
        _edge_mlp_body,
        grid=(E // _BE,),
        in_specs=[
            pl.BlockSpec((_BE, DE), lambda i: (i, 0)),
            pl.BlockSpec((DE, D), lambda i: (0, 0)),
            pl.BlockSpec((1, D), lambda i: (0, 0)),
        ],
        out_specs=pl.BlockSpec((2, _BE, 128), lambda i: (0, i, 0)),
        out_shape=jax.ShapeDtypeStruct((2, E, 128), _F32),
    )(ea, w, b)


_CH = 80
_EPT = E // 16
_NCHUNK = _EPT // _CH
_NPT = 624
_ZROWS = 208


def _sc_gather_scatter(x2, e2, src, dst):
    mesh = plsc.VectorSubcoreMesh(core_axis_name="c", subcore_axis_name="s")

    @functools.partial(
        pl.kernel,
        out_type=jax.ShapeDtypeStruct((2 * N, 128), _F32),
        mesh=mesh,
        scratch_types=[
            pltpu.VMEM((_CH,), jnp.int32),
            pltpu.VMEM((_CH,), jnp.int32),
            pltpu.VMEM((_CH, 128), _F32),
            pltpu.VMEM((_CH, 128), _F32),
            pltpu.VMEM((_ZROWS, 128), _F32),
            pltpu.VMEM_SHARED((N, 128), _F32),
            pltpu.SemaphoreType.DMA,
        ],
    )
    def body(x2_hbm, e2_hbm, src_hbm, dst_hbm, out_hbm,
             idx_s, idx_d, xbuf, ebuf, zbuf, agg_sh, sem):
        c = lax.axis_index("c")
        s = lax.axis_index("s")

        def zrow(j, carry):
            for k in range(8):
                zbuf[j, pl.ds(k * 16, 16)] = jnp.zeros((16,), _F32)
            return carry

        lax.fori_loop(0, _ZROWS, zrow, 0)
        for m in range(3):
            pltpu.sync_copy(zbuf, agg_sh.at[pl.ds(s * _NPT + m * _ZROWS,
                                                  _ZROWS)])

        @pl.when(s == 15)
        def _():
            pltpu.sync_copy(zbuf.at[pl.ds(0, 16)],
                            agg_sh.at[pl.ds(16 * _NPT, 16)])

        plsc.subcore_barrier()

        c_off = c * N

        def chunk(i, carry):
            base = s * _EPT + i * _CH
            pltpu.sync_copy(src_hbm.at[pl.ds(base, _CH)], idx_s)
            pltpu.sync_copy(dst_hbm.at[pl.ds(base, _CH)], idx_d)
            for k in range(_CH // 16):
                sl = pl.ds(k * 16, 16)
                idx_s[sl] = idx_s[sl] + c_off
            pltpu.async_copy(x2_hbm.at[idx_s], xbuf, sem).wait()
            pltpu.sync_copy(e2_hbm.at[pl.ds(c * E + base, _CH)], ebuf)

            def row(j, rc):
                for k in range(8):
                    sl = pl.ds(k * 16, 16)
                    ebuf[j, sl] = jnp.maximum(xbuf[j, sl] + ebuf[j, sl], 0.0)
                return rc

            lax.fori_loop(0, _CH, row, 0)
            pltpu.sync_copy(ebuf, agg_sh.at[idx_d], add=True)
            return carry

        lax.fori_loop(0, _NCHUNK, chunk, 0)
        plsc.subcore_barrier()
        pltpu.sync_copy(agg_sh.at[pl.ds(s * _NPT, _NPT)],
                        out_hbm.at[pl.ds(c_off + s * _NPT, _NPT)])

        @pl.when(s == 15)
        def _():
            pltpu.sync_copy(agg_sh.at[pl.ds(16 * _NPT, 16)],
                            out_hbm.at[pl.ds(c_off + 16 * _NPT, 16)])

    return body(x2, e2, src, dst)


_BN = 2000


def _silu(v):
    return v * jax.nn.sigmoid(v)


def _node_mlp_body(agg_ref, x_ref, scale_ref, w1_ref, b1_ref, w2_ref, b2_ref,
                   oht_ref, h2_ref, st_ref):
    i = pl.program_id(0)
    h0 = agg_ref[...] + scale_ref[0, 0] * x_ref[...]
    h1 = _silu(jnp.dot(h0, w1_ref[...], preferred_element_type=_F32,
                       precision=_P) + b1_ref[...])
    h2 = _silu(jnp.dot(h1, w2_ref[...], preferred_element_type=_F32,
                       precision=_P) + b2_ref[...])
    h2_ref[...] = h2
    rs = jnp.sum(h2, axis=1)
    rs2 = jnp.sum(h2 * h2, axis=1)
    stacked = jnp.concatenate(
        [rs[None, :], rs2[None, :], jnp.ones((1, _BN), _F32),
         jnp.zeros((5, _BN), _F32)], axis=0)
    part = jnp.dot(stacked, oht_ref[...], preferred_element_type=_F32,
                   precision=_P)

    @pl.when(i == 0)
    def _():
        st_ref[...] = part

    @pl.when(i > 0)
    def _():
        st_ref[...] = st_ref[...] + part


def _node_mlp(agg, x, scale, w1, b1, w2, b2, oht):
    return pl.pallas_call(
        _node_mlp_body,
        grid=(N // _BN,),
        in_specs=[
            pl.BlockSpec((_BN, D), lambda i: (i, 0)),
            pl.BlockSpec((_BN, D), lambda i: (i, 0)),
            pl.BlockSpec((1, 1), lambda i: (0, 0)),
            pl.BlockSpec((D, D), lambda i: (0, 0)),
            pl.BlockSpec((1, D), lambda i: (0, 0)),
            pl.BlockSpec((D, D), lambda i: (0, 0)),
            pl.BlockSpec((1, D), lambda i: (0, 0)),
            pl.BlockSpec((_BN, 128), lambda i: (i, 0)),
        ],
        out_specs=[
            pl.BlockSpec((_BN, D), lambda i: (i, 0)),
            pl.BlockSpec((8, 128), lambda i: (0, 0)),
        ],
        out_shape=[
            jax.ShapeDtypeStruct((N, D), _F32),
            jax.ShapeDtypeStruct((8, 128), _F32),
        ],
    )(agg, x, scale, w1, b1, w2, b2, oht)


def _norm_body(h2_ref, x_ref, st_ref, oht_ref, w_ref, b_ref, out_ref):
    st = st_ref[...]
    cnt = jnp.maximum(st[2:3, :], 1.0)
    norm = cnt * float(D)
    mean = st[0:1, :] / norm
    var = st[1:2, :] / norm - mean * mean
    rstd = lax.rsqrt(var + 1e-5)
    oht = oht_ref[...]
    dn = (((1,), (1,)), ((), ()))
    m_n = lax.dot_general(oht, mean, dn, precision=_P,
                          preferred_element_type=_F32)
    r_n = lax.dot_general(oht, rstd, dn, precision=_P,
                          preferred_element_type=_F32)
    out = (h2_ref[...] - m_n) * r_n * w_ref[...] + b_ref[...]
    out = (out + x_ref[...]) * 0.5
    out_ref[...] = jnp.maximum(out, 0.0)


def _graph_norm(h2, x, st, oht, w, b):
    return pl.pallas_call(
        _norm_body,
        grid=(N // _BN,),
        in_specs=[
            pl.BlockSpec((_BN, D), lambda i: (i, 0)),
            pl.BlockSpec((_BN, D), lambda i: (i, 0)),
            pl.BlockSpec((8, 128), lambda i: (0, 0)),
            pl.BlockSpec((_BN, 128), lambda i: (i, 0)),
            pl.BlockSpec((1, D), lambda i: (0, 0)),
            pl.BlockSpec((1, D), lambda i: (0, 0)),
        ],
        out_specs=pl.BlockSpec((_BN, D), lambda i: (i, 0)),
        out_shape=jax.ShapeDtypeStruct((N, D), _F32),
    )(h2, x, st, oht, w, b)


def kernel(x, edge_index, edge_attr, node2graph,
           c1_edge_W, c1_edge_b, c1_W1, c1_b1, c1_W2, c1_b2, c1_eps,
           c2_edge_W, c2_edge_b, c2_W1, c2_b1, c2_W2, c2_b2, c2_eps,
           gn1_w, gn1_b, gn2_w, gn2_b):
    src = edge_index[0]
    dst = edge_index[1]

    e2 = _edge_mlp(edge_attr, c2_edge_W, c2_edge_b.reshape(1, D))
    e2 = e2.reshape(2 * E, 128)
    x2 = x.reshape(N, 2, 128).transpose(1, 0, 2).reshape(2 * N, 128)

    agg2 = _sc_gather_scatter(x2, e2, src, dst)
    agg = agg2.reshape(2, N, 128).transpose(1, 0, 2).reshape(N, D)

    oht = (node2graph[:, None] == jnp.arange(128, dtype=node2graph.dtype)
           [None, :]).astype(_F32)
    scale = (1.0 + c2_eps).reshape(1, 1)

    h2, st = _node_mlp(agg, x, scale, c2_W1, c2_b1.reshape(1, D),
                       c2_W2, c2_b2.reshape(1, D), oht)
    return _graph_norm(h2, x, st, oht, gn2_w.reshape(1, D),
                       gn2_b.reshape(1, D))


# speedup vs baseline: 3.5527x; 1.7864x over previous
import functools

import jax
import jax.numpy as jnp
from jax import lax
from jax.experimental import pallas as pl
from jax.experimental.pallas import tpu as pltpu
from jax.experimental.pallas import tpu_sc as plsc

N = 10000
E = 160000
D = 256
DE = 16
G = 64

_P = jax.lax.Precision.HIGHEST
_F32 = jnp.float32

_BE = 2000


def _edge_mlp_body(ea_ref, w_ref, b_ref, out_ref):
    e = jnp.dot(ea_ref[...], w_ref[...],
                preferred_element_type=_F32) + b_ref[...]
    e = e * jax.nn.sigmoid(e)
    out_ref[0] = e[:, :128]
    out_ref[1] = e[:, 128:]


def _edge_mlp(ea, w, b):
    return pl.pallas_call(
        _edge_mlp_body,
        grid=(E // _BE,),
        in_specs=[
            pl.BlockSpec((_BE, DE), lambda i: (i, 0)),
            pl.BlockSpec((DE, D), lambda i: (0, 0)),
            pl.BlockSpec((1, D), lambda i: (0, 0)),
        ],
        out_specs=pl.BlockSpec((2, _BE, 128), lambda i: (0, i, 0)),
        out_shape=jax.ShapeDtypeStruct((2, E, 128), _F32),
    )(ea, w, b)


_CH = 40
_EPT = E // 16
_NCHUNK = _EPT // _CH
_NPT = 624


def _sc_gather_scatter(x2, e2, src3, dst3):
    mesh = plsc.VectorSubcoreMesh(core_axis_name="c", subcore_axis_name="s")

    @functools.partial(
        pl.kernel,
        out_type=jax.ShapeDtypeStruct((2 * N, 128), _F32),
        mesh=mesh,
        scratch_types=[
            pltpu.VMEM((_CH,), jnp.int32),
            pltpu.VMEM((_CH,), jnp.int32),
            pltpu.VMEM((_CH,), jnp.int32),
            pltpu.VMEM((_CH,), jnp.int32),
            pltpu.VMEM((_CH,), jnp.int32),
            pltpu.VMEM((_CH,), jnp.int32),
            pltpu.VMEM((_CH, 128), _F32),
            pltpu.VMEM((_CH, 128), _F32),
            pltpu.VMEM((_CH, 128), _F32),
            pltpu.VMEM((_CH, 128), _F32),
            pltpu.VMEM_SHARED((N, 128), _F32),
            pltpu.SemaphoreType.DMA,
            pltpu.SemaphoreType.DMA,
            pltpu.SemaphoreType.DMA,
            pltpu.SemaphoreType.DMA,
            pltpu.SemaphoreType.DMA,
            pltpu.SemaphoreType.DMA,
            pltpu.SemaphoreType.DMA,
            pltpu.SemaphoreType.DMA,
            pltpu.SemaphoreType.DMA,
        ],
    )
    def body(x2_hbm, e2_hbm, src_hbm, dst_hbm, out_hbm,
             sidx0, sidx1, sidx2, didx0, didx1, didx2,
             xbuf0, xbuf1, ebuf0, ebuf1, agg_sh,
             isem0, isem1, isem2, gsem0, gsem1, esem0, esem1, ssem0, ssem1):
        c = lax.axis_index("c")
        s = lax.axis_index("s")
        sidx = [sidx0, sidx1, sidx2]
        didx = [didx0, didx1, didx2]
        isem = [isem0, isem1, isem2]
        xbuf = [xbuf0, xbuf1]
        ebuf = [ebuf0, ebuf1]
        gsem = [gsem0, gsem1]
        esem = [esem0, esem1]
        ssem = [ssem0, ssem1]

        srow = (c * 16 + s) * _NCHUNK
        drow = s * _NCHUNK

        def idx_issue(i, r):
            pltpu.async_copy(src_hbm.at[srow + i], sidx[r], isem[r])
            pltpu.async_copy(dst_hbm.at[drow + i], didx[r], isem[r])

        def idx_wait(i, r):
            pltpu.make_async_copy(src_hbm.at[srow + i], sidx[r],
                                  isem[r]).wait()
            pltpu.make_async_copy(dst_hbm.at[drow + i], didx[r],
                                  isem[r]).wait()

        idx_issue(0, 0)
        idx_issue(1, 1)

        def zrow(j, carry):
            for k in range(8):
                ebuf0[j, pl.ds(k * 16, 16)] = jnp.zeros((16,), _F32)
            return carry

        lax.fori_loop(0, _CH, zrow, 0)
        for m in range(15):
            pltpu.sync_copy(ebuf0, agg_sh.at[pl.ds(s * _NPT + m * _CH,
                                                   _CH)])
        pltpu.sync_copy(ebuf0.at[pl.ds(0, 24)],
                        agg_sh.at[pl.ds(s * _NPT + 15 * _CH, 24)])

        @pl.when(s == 15)
        def _():
            pltpu.sync_copy(ebuf0.at[pl.ds(0, 16)],
                            agg_sh.at[pl.ds(16 * _NPT, 16)])

        plsc.subcore_barrier()

        def issue(i, p, r):
            pltpu.async_copy(x2_hbm.at[sidx[r]], xbuf[p], gsem[p])
            pltpu.async_copy(e2_hbm.at[pl.ds(c * E + s * _EPT + i * _CH,
                                             _CH)], ebuf[p], esem[p])

        idx_wait(0, 0)
        issue(0, 0, 0)

        def step(i, p, r):
            q = 1 - p
            r1 = (r + 1) % 3
            r2 = (r + 2) % 3

            @pl.when(i >= 1)
            def _():
                pltpu.make_async_copy(
                    ebuf[q], agg_sh.at[didx[r2]], ssem[q]).wait()

            @pl.when(i + 2 < _NCHUNK)
            def _():
                idx_issue(i + 2, r2)

            @pl.when(i + 1 < _NCHUNK)
            def _():
                idx_wait(i + 1, r1)
                issue(i + 1, q, r1)

            pltpu.make_async_copy(x2_hbm.at[sidx[r]], xbuf[p],
                                  gsem[p]).wait()
            pltpu.make_async_copy(
                e2_hbm.at[pl.ds(c * E + s * _EPT + i * _CH, _CH)],
                ebuf[p], esem[p]).wait()

            def row(j, rc):
                for k in range(8):
                    sl = pl.ds(k * 16, 16)
                    ebuf[p][j, sl] = jnp.maximum(
                        xbuf[p][j, sl] + ebuf[p][j, sl], 0.0)
                return rc

            lax.fori_loop(0, _CH, row, 0)
            pltpu.async_copy(ebuf[p], agg_sh.at[didx[r]], ssem[p],
                             add=True)

        def chunk(i, carry):
            for pp in range(2):
                for rr in range(3):
                    @pl.when(jnp.logical_and(i % 2 == pp, i % 3 == rr))
                    def _():
                        step(i, pp, rr)

            return carry

        lax.fori_loop(0, _NCHUNK, chunk, 0)
        pltpu.make_async_copy(
            ebuf[1], agg_sh.at[didx[(_NCHUNK - 1) % 3]], ssem[1]).wait()
        plsc.subcore_barrier()
        pltpu.sync_copy(agg_sh.at[pl.ds(s * _NPT, _NPT)],
                        out_hbm.at[pl.ds(c * N + s * _NPT, _NPT)])

        @pl.when(s == 15)
        def _():
            pltpu.sync_copy(agg_sh.at[pl.ds(16 * _NPT, 16)],
                            out_hbm.at[pl.ds(c * N + 16 * _NPT, 16)])

    return body(x2, e2, src3, dst3)


_BN = 2000


def _silu(v):
    return v * jax.nn.sigmoid(v)


def _node_mlp_body(agg_ref, x_ref, scale_ref, w1_ref, b1_ref, w2_ref, b2_ref,
                   oht_ref, h2_ref, st_ref):
    i = pl.program_id(0)
    h0 = agg_ref[...] + scale_ref[0, 0] * x_ref[...]
    h1 = _silu(jnp.dot(h0, w1_ref[...],
                       preferred_element_type=_F32) + b1_ref[...])
    h2 = _silu(jnp.dot(h1, w2_ref[...],
                       preferred_element_type=_F32) + b2_ref[...])
    h2_ref[...] = h2
    rs = jnp.sum(h2, axis=1)
    rs2 = jnp.sum(h2 * h2, axis=1)
    stacked = jnp.concatenate(
        [rs[None, :], rs2[None, :], jnp.ones((1, _BN), _F32),
         jnp.zeros((5, _BN), _F32)], axis=0)
    part = jnp.dot(stacked, oht_ref[...], preferred_element_type=_F32,
                   precision=_P)

    @pl.when(i == 0)
    def _():
        st_ref[...] = part

    @pl.when(i > 0)
    def _():
        st_ref[...] = st_ref[...] + part


def _node_mlp(agg, x, scale, w1, b1, w2, b2, oht):
    return pl.pallas_call(
        _node_mlp_body,
        grid=(N // _BN,),
        in_specs=[
            pl.BlockSpec((_BN, D), lambda i: (i, 0)),
            pl.BlockSpec((_BN, D), lambda i: (i, 0)),
            pl.BlockSpec((1, 1), lambda i: (0, 0)),
            pl.BlockSpec((D, D), lambda i: (0, 0)),
            pl.BlockSpec((1, D), lambda i: (0, 0)),
            pl.BlockSpec((D, D), lambda i: (0, 0)),
            pl.BlockSpec((1, D), lambda i: (0, 0)),
            pl.BlockSpec((_BN, 128), lambda i: (i, 0)),
        ],
        out_specs=[
            pl.BlockSpec((_BN, D), lambda i: (i, 0)),
            pl.BlockSpec((8, 128), lambda i: (0, 0)),
        ],
        out_shape=[
            jax.ShapeDtypeStruct((N, D), _F32),
            jax.ShapeDtypeStruct((8, 128), _F32),
        ],
    )(agg, x, scale, w1, b1, w2, b2, oht)


def _norm_body(h2_ref, x_ref, st_ref, oht_ref, w_ref, b_ref, out_ref):
    st = st_ref[...]
    cnt = jnp.maximum(st[2:3, :], 1.0)
    norm = cnt * float(D)
    mean = st[0:1, :] / norm
    var = st[1:2, :] / norm - mean * mean
    rstd = lax.rsqrt(var + 1e-5)
    oht = oht_ref[...]
    dn = (((1,), (1,)), ((), ()))
    m_n = lax.dot_general(oht, mean, dn, precision=_P,
                          preferred_element_type=_F32)
    r_n = lax.dot_general(oht, rstd, dn, precision=_P,
                          preferred_element_type=_F32)
    out = (h2_ref[...] - m_n) * r_n * w_ref[...] + b_ref[...]
    out = (out + x_ref[...]) * 0.5
    out_ref[...] = jnp.maximum(out, 0.0)


def _graph_norm(h2, x, st, oht, w, b):
    return pl.pallas_call(
        _norm_body,
        grid=(N // _BN,),
        in_specs=[
            pl.BlockSpec((_BN, D), lambda i: (i, 0)),
            pl.BlockSpec((_BN, D), lambda i: (i, 0)),
            pl.BlockSpec((8, 128), lambda i: (0, 0)),
            pl.BlockSpec((_BN, 128), lambda i: (i, 0)),
            pl.BlockSpec((1, D), lambda i: (0, 0)),
            pl.BlockSpec((1, D), lambda i: (0, 0)),
        ],
        out_specs=pl.BlockSpec((_BN, D), lambda i: (i, 0)),
        out_shape=jax.ShapeDtypeStruct((N, D), _F32),
    )(h2, x, st, oht, w, b)


def kernel(x, edge_index, edge_attr, node2graph,
           c1_edge_W, c1_edge_b, c1_W1, c1_b1, c1_W2, c1_b2, c1_eps,
           c2_edge_W, c2_edge_b, c2_W1, c2_b1, c2_W2, c2_b2, c2_eps,
           gn1_w, gn1_b, gn2_w, gn2_b):
    srcr = edge_index[0].reshape(16 * _NCHUNK, _CH)
    src3 = jnp.concatenate([srcr, srcr + N], axis=0)
    dst3 = edge_index[1].reshape(16 * _NCHUNK, _CH)

    e2 = _edge_mlp(edge_attr, c2_edge_W, c2_edge_b.reshape(1, D))
    e2 = e2.reshape(2 * E, 128)
    x2 = x.reshape(N, 2, 128).transpose(1, 0, 2).reshape(2 * N, 128)

    agg2 = _sc_gather_scatter(x2, e2, src3, dst3)
    agg = agg2.reshape(2, N, 128).transpose(1, 0, 2).reshape(N, D)

    oht = (node2graph[:, None] == jnp.arange(128, dtype=node2graph.dtype)
           [None, :]).astype(_F32)
    scale = (1.0 + c2_eps).reshape(1, 1)

    h2, st = _node_mlp(agg, x, scale, c2_W1, c2_b1.reshape(1, D),
                       c2_W2, c2_b2.reshape(1, D), oht)
    return _graph_norm(h2, x, st, oht, gn2_w.reshape(1, D),
                       gn2_b.reshape(1, D))


# speedup vs baseline: 3.7706x; 1.0613x over previous
import functools

import jax
import jax.numpy as jnp
from jax import lax
from jax.experimental import pallas as pl
from jax.experimental.pallas import tpu as pltpu
from jax.experimental.pallas import tpu_sc as plsc

N = 10000
E = 160000
D = 256
DE = 16
G = 64

_P = jax.lax.Precision.HIGHEST
_F32 = jnp.float32

_BE = 2000


def _edge_mlp_body(ea_ref, w_ref, b_ref, out_ref):
    e = jnp.dot(ea_ref[...], w_ref[...],
                preferred_element_type=_F32) + b_ref[...]
    e = e * jax.nn.sigmoid(e)
    out_ref[0] = e[:, :128]
    out_ref[1] = e[:, 128:]


def _edge_mlp(ea, w, b):
    return pl.pallas_call(
        _edge_mlp_body,
        grid=(E // _BE,),
        in_specs=[
            pl.BlockSpec((_BE, DE), lambda i: (i, 0)),
            pl.BlockSpec((DE, D), lambda i: (0, 0)),
            pl.BlockSpec((1, D), lambda i: (0, 0)),
        ],
        out_specs=pl.BlockSpec((2, _BE, 128), lambda i: (0, i, 0)),
        out_shape=jax.ShapeDtypeStruct((2, E, 128), _F32),
    )(ea, w, b)


_CH = 40
_EPT = E // 16
_NCHUNK = _EPT // _CH
_NPT = 624


def _sc_gather_scatter(x2, e2, src3, dst3):
    mesh = plsc.VectorSubcoreMesh(core_axis_name="c", subcore_axis_name="s")

    @functools.partial(
        pl.kernel,
        out_type=jax.ShapeDtypeStruct((2 * N, 128), _F32),
        mesh=mesh,
        scratch_types=[
            pltpu.VMEM((_CH,), jnp.int32),
            pltpu.VMEM((_CH,), jnp.int32),
            pltpu.VMEM((_CH,), jnp.int32),
            pltpu.VMEM((_CH,), jnp.int32),
            pltpu.VMEM((_CH,), jnp.int32),
            pltpu.VMEM((_CH,), jnp.int32),
            pltpu.VMEM((_CH, 128), _F32),
            pltpu.VMEM((_CH, 128), _F32),
            pltpu.VMEM((_CH, 128), _F32),
            pltpu.VMEM((_CH, 128), _F32),
            pltpu.VMEM_SHARED((N, 128), _F32),
            pltpu.SemaphoreType.DMA,
            pltpu.SemaphoreType.DMA,
            pltpu.SemaphoreType.DMA,
            pltpu.SemaphoreType.DMA,
            pltpu.SemaphoreType.DMA,
            pltpu.SemaphoreType.DMA,
            pltpu.SemaphoreType.DMA,
            pltpu.SemaphoreType.DMA,
            pltpu.SemaphoreType.DMA,
        ],
    )
    def body(x2_hbm, e2_hbm, src_hbm, dst_hbm, out_hbm,
             sidx0, sidx1, sidx2, didx0, didx1, didx2,
             xbuf0, xbuf1, ebuf0, ebuf1, agg_sh,
             isem0, isem1, isem2, gsem0, gsem1, esem0, esem1, ssem0, ssem1):
        c = lax.axis_index("c")
        s = lax.axis_index("s")
        sidx = [sidx0, sidx1, sidx2]
        didx = [didx0, didx1, didx2]
        isem = [isem0, isem1, isem2]
        xbuf = [xbuf0, xbuf1]
        ebuf = [ebuf0, ebuf1]
        gsem = [gsem0, gsem1]
        esem = [esem0, esem1]
        ssem = [ssem0, ssem1]

        srow = (c * 16 + s) * _NCHUNK
        drow = s * _NCHUNK

        def idx_issue(i, r):
            pltpu.async_copy(src_hbm.at[srow + i], sidx[r], isem[r])
            pltpu.async_copy(dst_hbm.at[drow + i], didx[r], isem[r])

        def idx_wait(i, r):
            pltpu.make_async_copy(src_hbm.at[srow + i], sidx[r],
                                  isem[r]).wait()
            pltpu.make_async_copy(dst_hbm.at[drow + i], didx[r],
                                  isem[r]).wait()

        idx_issue(0, 0)
        idx_issue(1, 1)

        def zrow(j, carry):
            for k in range(8):
                ebuf0[j, pl.ds(k * 16, 16)] = jnp.zeros((16,), _F32)
            return carry

        lax.fori_loop(0, _CH, zrow, 0)
        for m in range(15):
            pltpu.sync_copy(ebuf0, agg_sh.at[pl.ds(s * _NPT + m * _CH,
                                                   _CH)])
        pltpu.sync_copy(ebuf0.at[pl.ds(0, 24)],
                        agg_sh.at[pl.ds(s * _NPT + 15 * _CH, 24)])

        @pl.when(s == 15)
        def _():
            pltpu.sync_copy(ebuf0.at[pl.ds(0, 16)],
                            agg_sh.at[pl.ds(16 * _NPT, 16)])

        plsc.subcore_barrier()

        def issue(i, p, r):
            pltpu.async_copy(x2_hbm.at[sidx[r]], xbuf[p], gsem[p])
            pltpu.async_copy(e2_hbm.at[pl.ds(c * E + s * _EPT + i * _CH,
                                             _CH)], ebuf[p], esem[p])

        idx_wait(0, 0)
        issue(0, 0, 0)

        def step(i, p, r):
            q = 1 - p
            r1 = (r + 1) % 3
            r2 = (r + 2) % 3

            @pl.when(i >= 1)
            def _():
                pltpu.make_async_copy(
                    ebuf[q], agg_sh.at[didx[r2]], ssem[q]).wait()

            @pl.when(i + 2 < _NCHUNK)
            def _():
                idx_issue(i + 2, r2)

            @pl.when(i + 1 < _NCHUNK)
            def _():
                idx_wait(i + 1, r1)
                issue(i + 1, q, r1)

            pltpu.make_async_copy(x2_hbm.at[sidx[r]], xbuf[p],
                                  gsem[p]).wait()
            pltpu.make_async_copy(
                e2_hbm.at[pl.ds(c * E + s * _EPT + i * _CH, _CH)],
                ebuf[p], esem[p]).wait()

            def row(j, rc):
                for k in range(8):
                    sl = pl.ds(k * 16, 16)
                    ebuf[p][j, sl] = jnp.maximum(
                        xbuf[p][j, sl] + ebuf[p][j, sl], 0.0)
                return rc

            lax.fori_loop(0, _CH, row, 0)
            pltpu.async_copy(ebuf[p], agg_sh.at[didx[r]], ssem[p],
                             add=True)

        def chunk(i, carry):
            for pp in range(2):
                for rr in range(3):
                    @pl.when(jnp.logical_and(i % 2 == pp, i % 3 == rr))
                    def _():
                        step(i, pp, rr)

            return carry

        lax.fori_loop(0, _NCHUNK, chunk, 0)
        pltpu.make_async_copy(
            ebuf[1], agg_sh.at[didx[(_NCHUNK - 1) % 3]], ssem[1]).wait()
        plsc.subcore_barrier()
        pltpu.sync_copy(agg_sh.at[pl.ds(s * _NPT, _NPT)],
                        out_hbm.at[pl.ds(c * N + s * _NPT, _NPT)])

        @pl.when(s == 15)
        def _():
            pltpu.sync_copy(agg_sh.at[pl.ds(16 * _NPT, 16)],
                            out_hbm.at[pl.ds(c * N + 16 * _NPT, 16)])

    return body(x2, e2, src3, dst3)


_BN = 2000
_NB = N // _BN


def _silu(v):
    return v * jax.nn.sigmoid(v)


def _xsplit_body(x_ref, out_ref):
    xv = x_ref[...]
    out_ref[0] = xv[:, :128]
    out_ref[1] = xv[:, 128:]


def _xsplit(x):
    return pl.pallas_call(
        _xsplit_body,
        grid=(_NB,),
        in_specs=[pl.BlockSpec((_BN, D), lambda i: (i, 0))],
        out_specs=pl.BlockSpec((2, _BN, 128), lambda i: (0, i, 0)),
        out_shape=jax.ShapeDtypeStruct((2, N, 128), _F32),
    )(x)


def _oht(n2g_ref):
    n2g = n2g_ref[0, 0, :]
    lanes = jax.lax.broadcasted_iota(jnp.int32, (_BN, 128), 1)
    return (n2g[:, None] == lanes).astype(_F32)


def _node_body(lo_ref, hi_ref, x_ref, scale_ref, w1_ref, b1_ref, w2_ref,
               b2_ref, n2g_ref, gw_ref, gb_ref, out_ref, h2_s, st_s):
    i = pl.program_id(0)

    @pl.when(i < _NB)
    def _():
        ib = i
        h0 = jnp.concatenate([lo_ref[...], hi_ref[...]], axis=1)
        h0 = h0 + scale_ref[0, 0] * x_ref[...]
        h1 = _silu(jnp.dot(h0, w1_ref[...],
                           preferred_element_type=_F32) + b1_ref[...])
        h2 = _silu(jnp.dot(h1, w2_ref[...],
                           preferred_element_type=_F32) + b2_ref[...])
        h2_s[pl.ds(ib * _BN, _BN), :] = h2
        out_ref[...] = h2
        rs = jnp.sum(h2, axis=1)
        rs2 = jnp.sum(h2 * h2, axis=1)
        stacked = jnp.concatenate(
            [rs[None, :], rs2[None, :], jnp.ones((1, _BN), _F32),
             jnp.zeros((5, _BN), _F32)], axis=0)
        part = jnp.dot(stacked, _oht(n2g_ref), preferred_element_type=_F32,
                       precision=_P)

        @pl.when(i == 0)
        def _():
            st_s[...] = part

        @pl.when(i > 0)
        def _():
            st_s[...] = st_s[...] + part

    @pl.when(i >= _NB)
    def _():
        ib = i - _NB
        st = st_s[...]
        cnt = jnp.maximum(st[2:3, :], 1.0)
        norm = cnt * float(D)
        mean = st[0:1, :] / norm
        var = st[1:2, :] / norm - mean * mean
        rstd = lax.rsqrt(var + 1e-5)
        oht = _oht(n2g_ref)
        dn = (((1,), (1,)), ((), ()))
        m_n = lax.dot_general(oht, mean, dn, precision=_P,
                              preferred_element_type=_F32)
        r_n = lax.dot_general(oht, rstd, dn, precision=_P,
                              preferred_element_type=_F32)
        h2v = h2_s[pl.ds(ib * _BN, _BN), :]
        out = (h2v - m_n) * r_n * gw_ref[...] + gb_ref[...]
        out = (out + x_ref[...]) * 0.5
        out_ref[...] = jnp.maximum(out, 0.0)


def _node_stage(agg2, x, scale, w1, b1, w2, b2, n2g3, gw, gb):
    return pl.pallas_call(
        _node_body,
        grid=(2 * _NB,),
        in_specs=[
            pl.BlockSpec((_BN, 128), lambda i: (i % _NB, 0)),
            pl.BlockSpec((_BN, 128), lambda i: (_NB + i % _NB, 0)),
            pl.BlockSpec((_BN, D), lambda i: (i % _NB, 0)),
            pl.BlockSpec((1, 1), lambda i: (0, 0)),
            pl.BlockSpec((D, D), lambda i: (0, 0)),
            pl.BlockSpec((1, D), lambda i: (0, 0)),
            pl.BlockSpec((D, D), lambda i: (0, 0)),
            pl.BlockSpec((1, D), lambda i: (0, 0)),
            pl.BlockSpec((1, 1, _BN), lambda i: (i % _NB, 0, 0)),
            pl.BlockSpec((1, D), lambda i: (0, 0)),
            pl.BlockSpec((1, D), lambda i: (0, 0)),
        ],
        out_specs=pl.BlockSpec((_BN, D), lambda i: (i % _NB, 0)),
        out_shape=jax.ShapeDtypeStruct((N, D), _F32),
        scratch_shapes=[
            pltpu.VMEM((N, D), _F32),
            pltpu.VMEM((8, 128), _F32),
        ],
    )(agg2, agg2, x, scale, w1, b1, w2, b2, n2g3, gw, gb)


def kernel(x, edge_index, edge_attr, node2graph,
           c1_edge_W, c1_edge_b, c1_W1, c1_b1, c1_W2, c1_b2, c1_eps,
           c2_edge_W, c2_edge_b, c2_W1, c2_b1, c2_W2, c2_b2, c2_eps,
           gn1_w, gn1_b, gn2_w, gn2_b):
    srcr = edge_index[0].reshape(16 * _NCHUNK, _CH)
    src3 = jnp.concatenate([srcr, srcr + N], axis=0)
    dst3 = edge_index[1].reshape(16 * _NCHUNK, _CH)

    e2 = _edge_mlp(edge_attr, c2_edge_W, c2_edge_b.reshape(1, D))
    e2 = e2.reshape(2 * E, 128)
    x2 = _xsplit(x).reshape(2 * N, 128)

    agg2 = _sc_gather_scatter(x2, e2, src3, dst3)

    n2g3 = node2graph.reshape(_NB, 1, _BN)
    scale = (1.0 + c2_eps).reshape(1, 1)

    return _node_stage(agg2, x, scale, c2_W1, c2_b1.reshape(1, D),
                       c2_W2, c2_b2.reshape(1, D), n2g3,
                       gn2_w.reshape(1, D), gn2_b.reshape(1, D))
